# Initial kernel scaffold; baseline (speedup 1.0000x reference)
import jax
import jax.numpy as jnp
from jax.experimental import pallas as pl


def kernel(x, emb, W, b):
    raise NotImplementedError("write your pallas kernel here")


# speedup vs baseline: 3.1058x; 3.1058x over previous
import functools

import jax
import jax.numpy as jnp
from jax import lax
from jax.experimental import pallas as pl
from jax.experimental.pallas import tpu as pltpu
from jax.experimental.pallas import tpu_sc as plsc

VOCAB = 10000
DIM = 128
N_OUT = 128

_ROW_BLOCK = 1000


def _table_body(emb_ref, w_ref, b_ref, out_ref):
    acc = lax.dot_general(
        emb_ref[...], w_ref[...],
        dimension_numbers=(((1,), (1,)), ((), ())),
        preferred_element_type=jnp.float32,
    )
    out_ref[...] = acc + b_ref[...]


def _build_table(emb, W, b):
    grid = VOCAB // _ROW_BLOCK
    return pl.pallas_call(
        _table_body,
        grid=(grid,),
        in_specs=[
            pl.BlockSpec((_ROW_BLOCK, DIM), lambda i: (i, 0)),
            pl.BlockSpec((N_OUT, DIM), lambda i: (0, 0)),
            pl.BlockSpec((1, N_OUT), lambda i: (0, 0)),
        ],
        out_specs=pl.BlockSpec((_ROW_BLOCK, N_OUT), lambda i: (i, 0)),
        out_shape=jax.ShapeDtypeStruct((VOCAB, N_OUT), jnp.float32),
    )(emb, W, b.reshape(1, N_OUT))


@functools.cache
def _make_gather(n_idx):
    NC, NS = 2, 16
    NW = NC * NS
    b_per_w = n_idx // NW
    chunk = 800
    n_chunks = b_per_w // chunk
    mesh = plsc.VectorSubcoreMesh(core_axis_name="c", subcore_axis_name="s")

    @functools.partial(
        pl.kernel,
        mesh=mesh,
        out_type=jax.ShapeDtypeStruct((n_idx, N_OUT), jnp.float32),
        scratch_types=[
            pltpu.VMEM((b_per_w,), jnp.int32),
            pltpu.VMEM((chunk, N_OUT), jnp.float32),
            pltpu.SemaphoreType.DMA,
        ],
    )
    def gather_k(table_hbm, idx_hbm, out_hbm, idx_v, rows_v, sem):
        wid = lax.axis_index("s") * NC + lax.axis_index("c")
        base = pl.multiple_of(wid * b_per_w, 8)
        pltpu.sync_copy(idx_hbm.at[pl.ds(base, b_per_w)], idx_v)

        def body(i, carry):
            off = pl.multiple_of(i * chunk, 8)
            pltpu.async_copy(
                table_hbm.at[idx_v.at[pl.ds(off, chunk)]], rows_v, sem
            ).wait()
            pltpu.sync_copy(rows_v, out_hbm.at[pl.ds(base + off, chunk)])
            return carry

        lax.fori_loop(0, n_chunks, body, 0)

    return gather_k


def kernel(x, emb, W, b):
    table = _build_table(emb, W, b)
    idx = x.reshape(-1).astype(jnp.int32)
    out = _make_gather(idx.shape[0])(table, idx)
    return out.reshape(x.shape[0], x.shape[1], N_OUT)


# speedup vs baseline: 3.1221x; 1.0053x over previous
import functools

import jax
import jax.numpy as jnp
from jax import lax
from jax.experimental import pallas as pl
from jax.experimental.pallas import tpu as pltpu
from jax.experimental.pallas import tpu_sc as plsc

VOCAB = 10000
DIM = 128
N_OUT = 128

_ROW_BLOCK = 1000


def _table_body(emb_ref, w_ref, b_ref, out_ref):
    acc = lax.dot_general(
        emb_ref[...], w_ref[...],
        dimension_numbers=(((1,), (1,)), ((), ())),
        preferred_element_type=jnp.float32,
    )
    out_ref[...] = acc + b_ref[...]


def _build_table(emb, W, b):
    grid = VOCAB // _ROW_BLOCK
    return pl.pallas_call(
        _table_body,
        grid=(grid,),
        in_specs=[
            pl.BlockSpec((_ROW_BLOCK, DIM), lambda i: (i, 0)),
            pl.BlockSpec((N_OUT, DIM), lambda i: (0, 0)),
            pl.BlockSpec((1, N_OUT), lambda i: (0, 0)),
        ],
        out_specs=pl.BlockSpec((_ROW_BLOCK, N_OUT), lambda i: (i, 0)),
        out_shape=jax.ShapeDtypeStruct((VOCAB, N_OUT), jnp.float32),
    )(emb, W, b.reshape(1, N_OUT))


@functools.cache
def _make_gather(n_idx):
    NC, NS = 2, 16
    NW = NC * NS
    b_per_w = n_idx // NW
    chunk = 400
    nbuf = 2
    n_chunks = b_per_w // chunk
    mesh = plsc.VectorSubcoreMesh(core_axis_name="c", subcore_axis_name="s")

    @functools.partial(
        pl.kernel,
        mesh=mesh,
        out_type=jax.ShapeDtypeStruct((n_idx, N_OUT), jnp.float32),
        scratch_types=[
            pltpu.VMEM((b_per_w,), jnp.int32),
            *[pltpu.VMEM((chunk, N_OUT), jnp.float32) for _ in range(nbuf)],
            *[pltpu.SemaphoreType.DMA for _ in range(2 * nbuf)],
        ],
    )
    def gather_k(table_hbm, idx_hbm, out_hbm, idx_v, *bufs_and_sems):
        rows = bufs_and_sems[:nbuf]
        gsem = bufs_and_sems[nbuf:2 * nbuf]
        wsem = bufs_and_sems[2 * nbuf:]
        wid = lax.axis_index("s") * NC + lax.axis_index("c")
        base = pl.multiple_of(wid * b_per_w, 8)
        pltpu.sync_copy(idx_hbm.at[pl.ds(base, b_per_w)], idx_v)

        def gather_chunk(c, b):
            off = pl.multiple_of(c * chunk, 8)
            return pltpu.make_async_copy(
                table_hbm.at[idx_v.at[pl.ds(off, chunk)]], rows[b], gsem[b]
            )

        def write_chunk(c, b):
            off = pl.multiple_of(base + c * chunk, 8)
            return pltpu.make_async_copy(
                rows[b], out_hbm.at[pl.ds(off, chunk)], wsem[b]
            )

        for b in range(nbuf):
            gather_chunk(b, b).start()
        for c in range(n_chunks):
            b = c % nbuf
            gather_chunk(c, b).wait()
            write_chunk(c, b).start()
            if c + nbuf < n_chunks:
                write_chunk(c, b).wait()
                gather_chunk(c + nbuf, b).start()
        for c in range(max(0, n_chunks - nbuf), n_chunks):
            write_chunk(c, c % nbuf).wait()

    return gather_k


def kernel(x, emb, W, b):
    table = _build_table(emb, W, b)
    idx = x.reshape(-1).astype(jnp.int32)
    out = _make_gather(idx.shape[0])(table, idx)
    return out.reshape(x.shape[0], x.shape[1], N_OUT)


# speedup vs baseline: 5.2919x; 1.6950x over previous
import functools

import jax
import jax.numpy as jnp
from jax import lax
from jax.experimental import pallas as pl
from jax.experimental.pallas import tpu as pltpu
from jax.experimental.pallas import tpu_sc as plsc

VOCAB = 10000
DIM = 128
N_OUT = 128

_ROW_BLOCK = 1000


def _table_body(emb_ref, w_ref, b_ref, out_ref):
    acc = lax.dot_general(
        emb_ref[...], w_ref[...],
        dimension_numbers=(((1,), (1,)), ((), ())),
        preferred_element_type=jnp.float32,
    )
    out_ref[...] = acc + b_ref[...]


def _build_table(emb, W, b):
    grid = VOCAB // _ROW_BLOCK
    return pl.pallas_call(
        _table_body,
        grid=(grid,),
        in_specs=[
            pl.BlockSpec((_ROW_BLOCK, DIM), lambda i: (i, 0)),
            pl.BlockSpec((N_OUT, DIM), lambda i: (0, 0)),
            pl.BlockSpec((1, N_OUT), lambda i: (0, 0)),
        ],
        out_specs=pl.BlockSpec((_ROW_BLOCK, N_OUT), lambda i: (i, 0)),
        out_shape=jax.ShapeDtypeStruct((VOCAB, N_OUT), jnp.float32),
    )(emb, W, b.reshape(1, N_OUT))


@functools.cache
def _make_gather(n_batch, seq):
    NC, NS = 2, 16
    NW = NC * NS
    n_idx = n_batch * seq
    b_per_w = n_idx // NW
    chunk = 400
    nbuf = 2
    n_chunks = b_per_w // chunk
    bat_per_chunk = chunk // seq
    mesh = plsc.VectorSubcoreMesh(core_axis_name="c", subcore_axis_name="s")

    @functools.partial(
        pl.kernel,
        mesh=mesh,
        out_type=jax.ShapeDtypeStruct((n_batch, seq, N_OUT), jnp.float32),
        scratch_types=[
            pltpu.VMEM((b_per_w,), jnp.int32),
            *[pltpu.VMEM((chunk, N_OUT), jnp.float32) for _ in range(nbuf)],
            *[pltpu.SemaphoreType.DMA for _ in range(2 * nbuf)],
        ],
    )
    def gather_k(table_hbm, idx_hbm, out_hbm, idx_v, *bufs_and_sems):
        rows = bufs_and_sems[:nbuf]
        gsem = bufs_and_sems[nbuf:2 * nbuf]
        wsem = bufs_and_sems[2 * nbuf:]
        wid = lax.axis_index("s") * NC + lax.axis_index("c")
        base = pl.multiple_of(wid * b_per_w, 8)
        base_b = wid * (b_per_w // seq)
        pltpu.sync_copy(idx_hbm.at[pl.ds(base, b_per_w)], idx_v)

        def gather_chunk(c, b):
            off = pl.multiple_of(c * chunk, 8)
            return pltpu.make_async_copy(
                table_hbm.at[idx_v.at[pl.ds(off, chunk)]], rows[b], gsem[b]
            )

        def write_chunk(c, b):
            bo = base_b + c * bat_per_chunk
            return [
                pltpu.make_async_copy(
                    rows[b].at[pl.ds(k * seq, seq)], out_hbm.at[bo + k], wsem[b]
                )
                for k in range(bat_per_chunk)
            ]

        for b in range(nbuf):
            gather_chunk(b, b).start()
        for c in range(n_chunks):
            b = c % nbuf
            gather_chunk(c, b).wait()
            for cp in write_chunk(c, b):
                cp.start()
            if c + nbuf < n_chunks:
                for cp in write_chunk(c, b):
                    cp.wait()
                gather_chunk(c + nbuf, b).start()
        for c in range(max(0, n_chunks - nbuf), n_chunks):
            for cp in write_chunk(c, c % nbuf):
                cp.wait()

    return gather_k


def kernel(x, emb, W, b):
    table = _build_table(emb, W, b)
    idx = x.reshape(-1).astype(jnp.int32)
    return _make_gather(x.shape[0], x.shape[1])(table, idx)
